# initial kernel scaffold (unmeasured)
import functools

import jax
import jax.numpy as jnp
from jax import lax
from jax.experimental import pallas as pl
from jax.experimental.pallas import tpu as pltpu

N_DEV = 32
LOG2_N = 5


def kernel(x, w_mat):
    m_per, k = x.shape
    _, n_local = w_mat.shape
    m_total = m_per * N_DEV

    def body(x_ref, w_ref, out_ref, comm, amax_s, amax_r,
             send_sems, recv_sems, red_send, red_recv):
        my = lax.axis_index("i")
        left = lax.rem(my + N_DEV - 1, N_DEV)
        right = lax.rem(my + 1, N_DEV)

        barrier = pltpu.get_barrier_semaphore()
        pl.semaphore_signal(barrier, inc=1, device_id=(left,),
                            device_id_type=pl.DeviceIdType.MESH)
        pl.semaphore_signal(barrier, inc=1, device_id=(right,),
                            device_id_type=pl.DeviceIdType.MESH)
        pl.semaphore_wait(barrier, 2)

        w = w_ref[:, :]

        blk = jnp.dot(x_ref[:, :], w, preferred_element_type=jnp.float32)
        out_ref[pl.ds(my * m_per, m_per), :] = blk
        amax = jnp.max(jnp.abs(blk))

        for h in range(N_DEV - 1):
            src = x_ref if h == 0 else comm.at[h - 1]
            rdma = pltpu.make_async_remote_copy(
                src_ref=src,
                dst_ref=comm.at[h],
                send_sem=send_sems.at[h],
                recv_sem=recv_sems.at[h],
                device_id=(right,),
                device_id_type=pl.DeviceIdType.MESH,
            )
            rdma.start()
            rdma.wait()
            origin = lax.rem(my + N_DEV - 1 - h, N_DEV)
            blk = jnp.dot(comm[h], w, preferred_element_type=jnp.float32)
            out_ref[pl.ds(origin * m_per, m_per), :] = blk
            amax = jnp.maximum(amax, jnp.max(jnp.abs(blk)))

        for s in range(LOG2_N):
            amax_s[s] = jnp.full((8, 128), amax, jnp.float32)
            peer = my ^ (1 << s)
            rdma = pltpu.make_async_remote_copy(
                src_ref=amax_s.at[s],
                dst_ref=amax_r.at[s],
                send_sem=red_send.at[s],
                recv_sem=red_recv.at[s],
                device_id=(peer,),
                device_id_type=pl.DeviceIdType.MESH,
            )
            rdma.start()
            rdma.wait()
            amax = jnp.maximum(amax, amax_r[s, 0, 0])

        scale = amax / 127.0
        y = out_ref[:, :]
        q = jnp.clip(jnp.round(y / scale), -127.0, 127.0)
        out_ref[:, :] = q * scale

        @functools.partial(pl.run_scoped, exit_sem=pltpu.SemaphoreType.REGULAR)
        def _(exit_sem):
            pl.semaphore_signal(exit_sem, inc=1, device_id=(left,),
                                device_id_type=pl.DeviceIdType.MESH)
            pl.semaphore_signal(exit_sem, inc=1, device_id=(right,),
                                device_id_type=pl.DeviceIdType.MESH)
            pl.semaphore_wait(exit_sem, 2)

    return pl.pallas_call(
        body,
        out_shape=jax.ShapeDtypeStruct((m_total, n_local), jnp.float32),
        in_specs=[
            pl.BlockSpec(memory_space=pltpu.VMEM),
            pl.BlockSpec(memory_space=pltpu.VMEM),
        ],
        out_specs=pl.BlockSpec(memory_space=pltpu.VMEM),
        scratch_shapes=[
            pltpu.VMEM((N_DEV - 1, m_per, k), x.dtype),
            pltpu.VMEM((LOG2_N, 8, 128), jnp.float32),
            pltpu.VMEM((LOG2_N, 8, 128), jnp.float32),
            pltpu.SemaphoreType.DMA((N_DEV - 1,)),
            pltpu.SemaphoreType.DMA((N_DEV - 1,)),
            pltpu.SemaphoreType.DMA((LOG2_N,)),
            pltpu.SemaphoreType.DMA((LOG2_N,)),
        ],
        compiler_params=pltpu.CompilerParams(collective_id=0),
    )(x, w_mat)


# baseline (device time: 448444 ns/iter reference)
import functools

import jax
import jax.numpy as jnp
from jax import lax
from jax.experimental import pallas as pl
from jax.experimental.pallas import tpu as pltpu

N_DEV = 32
LOG2_N = 5


def kernel(x, w_mat):
    m_per, k = x.shape
    _, n_local = w_mat.shape
    m_total = m_per * N_DEV

    def body(x_ref, w_ref, out_ref, own, comm, amax_s, amax_r,
             send_sems, recv_sems, red_send, red_recv):
        my = lax.axis_index("i")
        left = lax.rem(my + N_DEV - 1, N_DEV)
        right = lax.rem(my + 1, N_DEV)

        barrier = pltpu.get_barrier_semaphore()
        pl.semaphore_signal(barrier, inc=1, device_id=(left,),
                            device_id_type=pl.DeviceIdType.MESH)
        pl.semaphore_signal(barrier, inc=1, device_id=(right,),
                            device_id_type=pl.DeviceIdType.MESH)
        pl.semaphore_wait(barrier, 2)

        w = w_ref[:, :].astype(jnp.bfloat16)
        own[:, :] = x_ref[:, :].astype(jnp.bfloat16)

        blk = jnp.dot(own[:, :], w, preferred_element_type=jnp.float32)
        out_ref[pl.ds(my * m_per, m_per), :] = blk
        amax = jnp.max(jnp.abs(blk))

        for h in range(N_DEV - 1):
            src = own if h == 0 else comm.at[h - 1]
            rdma = pltpu.make_async_remote_copy(
                src_ref=src,
                dst_ref=comm.at[h],
                send_sem=send_sems.at[h],
                recv_sem=recv_sems.at[h],
                device_id=(right,),
                device_id_type=pl.DeviceIdType.MESH,
            )
            rdma.start()
            rdma.wait()
            origin = lax.rem(my + N_DEV - 1 - h, N_DEV)
            blk = jnp.dot(comm[h], w, preferred_element_type=jnp.float32)
            out_ref[pl.ds(origin * m_per, m_per), :] = blk
            amax = jnp.maximum(amax, jnp.max(jnp.abs(blk)))

        for s in range(LOG2_N):
            amax_s[s] = jnp.full((8, 128), amax, jnp.float32)
            peer = my ^ (1 << s)
            rdma = pltpu.make_async_remote_copy(
                src_ref=amax_s.at[s],
                dst_ref=amax_r.at[s],
                send_sem=red_send.at[s],
                recv_sem=red_recv.at[s],
                device_id=(peer,),
                device_id_type=pl.DeviceIdType.MESH,
            )
            rdma.start()
            rdma.wait()
            amax = jnp.maximum(amax, amax_r[s, 0, 0])

        scale = amax / 127.0
        y = out_ref[:, :]
        q = jnp.clip(jnp.round(y / scale), -127.0, 127.0)
        out_ref[:, :] = q * scale

        @functools.partial(pl.run_scoped, exit_sem=pltpu.SemaphoreType.REGULAR)
        def _(exit_sem):
            pl.semaphore_signal(exit_sem, inc=1, device_id=(left,),
                                device_id_type=pl.DeviceIdType.MESH)
            pl.semaphore_signal(exit_sem, inc=1, device_id=(right,),
                                device_id_type=pl.DeviceIdType.MESH)
            pl.semaphore_wait(exit_sem, 2)

    return pl.pallas_call(
        body,
        out_shape=jax.ShapeDtypeStruct((m_total, n_local), jnp.float32),
        in_specs=[
            pl.BlockSpec(memory_space=pltpu.VMEM),
            pl.BlockSpec(memory_space=pltpu.VMEM),
        ],
        out_specs=pl.BlockSpec(memory_space=pltpu.VMEM),
        scratch_shapes=[
            pltpu.VMEM((m_per, k), jnp.bfloat16),
            pltpu.VMEM((N_DEV - 1, m_per, k), jnp.bfloat16),
            pltpu.VMEM((LOG2_N, 8, 128), jnp.float32),
            pltpu.VMEM((LOG2_N, 8, 128), jnp.float32),
            pltpu.SemaphoreType.DMA((N_DEV - 1,)),
            pltpu.SemaphoreType.DMA((N_DEV - 1,)),
            pltpu.SemaphoreType.DMA((LOG2_N,)),
            pltpu.SemaphoreType.DMA((LOG2_N,)),
        ],
        compiler_params=pltpu.CompilerParams(
            collective_id=0,
            vmem_limit_bytes=100 * 1024 * 1024,
            skip_device_barrier=True,
        ),
    )(x, w_mat)


# device time: 393230 ns/iter; 1.1404x vs baseline; 1.1404x over previous
import functools

import jax
import jax.numpy as jnp
from jax import lax
from jax.experimental import pallas as pl
from jax.experimental.pallas import tpu as pltpu

N_DEV = 32
LOG2_N = 5
HR = 16
HL = N_DEV - 1 - HR


def kernel(x, w_mat):
    m_per, k = x.shape
    _, n_local = w_mat.shape
    m_total = m_per * N_DEV

    def body(x_ref, w_ref, out_ref, own, comm_r, comm_l, amax_s, amax_r,
             r_send, r_recv, l_send, l_recv, red_send, red_recv):
        my = lax.axis_index("i")
        left = lax.rem(my + N_DEV - 1, N_DEV)
        right = lax.rem(my + 1, N_DEV)

        barrier = pltpu.get_barrier_semaphore()
        pl.semaphore_signal(barrier, inc=1, device_id=(left,),
                            device_id_type=pl.DeviceIdType.MESH)
        pl.semaphore_signal(barrier, inc=1, device_id=(right,),
                            device_id_type=pl.DeviceIdType.MESH)
        pl.semaphore_wait(barrier, 2)

        w = w_ref[:, :].astype(jnp.bfloat16)
        own[:, :] = x_ref[:, :].astype(jnp.bfloat16)

        r_rdma = [
            pltpu.make_async_remote_copy(
                src_ref=own if h == 0 else comm_r.at[h - 1],
                dst_ref=comm_r.at[h],
                send_sem=r_send.at[h],
                recv_sem=r_recv.at[h],
                device_id=(right,),
                device_id_type=pl.DeviceIdType.MESH,
            )
            for h in range(HR)
        ]
        l_rdma = [
            pltpu.make_async_remote_copy(
                src_ref=own if h == 0 else comm_l.at[h - 1],
                dst_ref=comm_l.at[h],
                send_sem=l_send.at[h],
                recv_sem=l_recv.at[h],
                device_id=(left,),
                device_id_type=pl.DeviceIdType.MESH,
            )
            for h in range(HL)
        ]

        r_rdma[0].start()
        l_rdma[0].start()

        blk = jnp.dot(own[:, :], w, preferred_element_type=jnp.float32)
        out_ref[pl.ds(my * m_per, m_per), :] = blk
        amax = jnp.max(jnp.abs(blk))

        for h in range(HR):
            r_rdma[h].wait_recv()
            if h + 1 < HR:
                r_rdma[h + 1].start()
            if h < HL:
                l_rdma[h].wait_recv()
                if h + 1 < HL:
                    l_rdma[h + 1].start()

            origin = lax.rem(my + N_DEV - 1 - h, N_DEV)
            blk = jnp.dot(comm_r[h], w, preferred_element_type=jnp.float32)
            out_ref[pl.ds(origin * m_per, m_per), :] = blk
            amax = jnp.maximum(amax, jnp.max(jnp.abs(blk)))
            if h < HL:
                origin = lax.rem(my + h + 1, N_DEV)
                blk = jnp.dot(comm_l[h], w, preferred_element_type=jnp.float32)
                out_ref[pl.ds(origin * m_per, m_per), :] = blk
                amax = jnp.maximum(amax, jnp.max(jnp.abs(blk)))

        for h in range(HR):
            r_rdma[h].wait_send()
        for h in range(HL):
            l_rdma[h].wait_send()

        for s in range(LOG2_N):
            amax_s[s] = jnp.full((8, 128), amax, jnp.float32)
            peer = my ^ (1 << s)
            rdma = pltpu.make_async_remote_copy(
                src_ref=amax_s.at[s],
                dst_ref=amax_r.at[s],
                send_sem=red_send.at[s],
                recv_sem=red_recv.at[s],
                device_id=(peer,),
                device_id_type=pl.DeviceIdType.MESH,
            )
            rdma.start()
            rdma.wait()
            amax = jnp.maximum(amax, amax_r[s, 0, 0])

        scale = amax / 127.0
        y = out_ref[:, :]
        q = jnp.clip(jnp.round(y / scale), -127.0, 127.0)
        out_ref[:, :] = q * scale

        @functools.partial(pl.run_scoped, exit_sem=pltpu.SemaphoreType.REGULAR)
        def _(exit_sem):
            pl.semaphore_signal(exit_sem, inc=1, device_id=(left,),
                                device_id_type=pl.DeviceIdType.MESH)
            pl.semaphore_signal(exit_sem, inc=1, device_id=(right,),
                                device_id_type=pl.DeviceIdType.MESH)
            pl.semaphore_wait(exit_sem, 2)

    return pl.pallas_call(
        body,
        out_shape=jax.ShapeDtypeStruct((m_total, n_local), jnp.float32),
        in_specs=[
            pl.BlockSpec(memory_space=pltpu.VMEM),
            pl.BlockSpec(memory_space=pltpu.VMEM),
        ],
        out_specs=pl.BlockSpec(memory_space=pltpu.VMEM),
        scratch_shapes=[
            pltpu.VMEM((m_per, k), jnp.bfloat16),
            pltpu.VMEM((HR, m_per, k), jnp.bfloat16),
            pltpu.VMEM((HL, m_per, k), jnp.bfloat16),
            pltpu.VMEM((LOG2_N, 8, 128), jnp.float32),
            pltpu.VMEM((LOG2_N, 8, 128), jnp.float32),
            pltpu.SemaphoreType.DMA((HR,)),
            pltpu.SemaphoreType.DMA((HR,)),
            pltpu.SemaphoreType.DMA((HL,)),
            pltpu.SemaphoreType.DMA((HL,)),
            pltpu.SemaphoreType.DMA((LOG2_N,)),
            pltpu.SemaphoreType.DMA((LOG2_N,)),
        ],
        compiler_params=pltpu.CompilerParams(
            collective_id=0,
            vmem_limit_bytes=100 * 1024 * 1024,
            skip_device_barrier=True,
        ),
    )(x, w_mat)


# device time: 225960 ns/iter; 1.9846x vs baseline; 1.7403x over previous
import functools

import jax
import jax.numpy as jnp
from jax import lax
from jax.experimental import pallas as pl
from jax.experimental.pallas import tpu as pltpu

N_DEV = 32
LOG2_N = 5
HR = 16
HL = N_DEV - 1 - HR


def kernel(x, w_mat):
    m_per, k = x.shape
    _, n_local = w_mat.shape
    m_total = m_per * N_DEV

    def body(x_ref, w_ref, out_ref, own_x, own_w, wcom_r, wcom_l,
             ysend, yrecv, amax_s, amax_r,
             r_send, r_recv, l_send, l_recv, y_send_sems, y_recv_sems,
             red_send, red_recv):
        my = lax.axis_index("i")
        left = lax.rem(my + N_DEV - 1, N_DEV)
        right = lax.rem(my + 1, N_DEV)

        barrier = pltpu.get_barrier_semaphore()
        pl.semaphore_signal(barrier, inc=1, device_id=(left,),
                            device_id_type=pl.DeviceIdType.MESH)
        pl.semaphore_signal(barrier, inc=1, device_id=(right,),
                            device_id_type=pl.DeviceIdType.MESH)
        pl.semaphore_wait(barrier, 2)

        own_x[:, :] = x_ref[:, :].astype(jnp.bfloat16)
        own_w[:, :] = jnp.swapaxes(w_ref[:, :], 0, 1).astype(jnp.bfloat16)
        xv = own_x[:, :]

        r_rdma = [
            pltpu.make_async_remote_copy(
                src_ref=own_w if h == 0 else wcom_r.at[h - 1],
                dst_ref=wcom_r.at[h],
                send_sem=r_send.at[h],
                recv_sem=r_recv.at[h],
                device_id=(right,),
                device_id_type=pl.DeviceIdType.MESH,
            )
            for h in range(HR)
        ]
        l_rdma = [
            pltpu.make_async_remote_copy(
                src_ref=own_w if h == 0 else wcom_l.at[h - 1],
                dst_ref=wcom_l.at[h],
                send_sem=l_send.at[h],
                recv_sem=l_recv.at[h],
                device_id=(left,),
                device_id_type=pl.DeviceIdType.MESH,
            )
            for h in range(HL)
        ]
        y_rdma = [None] * N_DEV
        for rho in range(1, N_DEV):
            y_rdma[rho] = pltpu.make_async_remote_copy(
                src_ref=ysend.at[rho],
                dst_ref=yrecv.at[rho],
                send_sem=y_send_sems.at[rho],
                recv_sem=y_recv_sems.at[rho],
                device_id=(lax.rem(my + rho, N_DEV),),
                device_id_type=pl.DeviceIdType.MESH,
            )

        pending_sends = []

        def drain_sends(before_hop):
            while pending_sends and pending_sends[0][0] < before_hop:
                pending_sends.pop(0)[1].wait_send()

        r_rdma[0].start()
        l_rdma[0].start()
        pending_sends += [(0, r_rdma[0]), (0, l_rdma[0])]

        blk = lax.dot_general(xv, own_w[:, :], (((1,), (1,)), ((), ())),
                              preferred_element_type=jnp.float32)
        out_ref[pl.ds(my * m_per, m_per), :] = blk
        amax = jnp.max(jnp.abs(blk))

        for h in range(HR):
            r_rdma[h].wait_recv()
            if h + 1 < HR:
                r_rdma[h + 1].start()
                pending_sends.append((h + 1, r_rdma[h + 1]))
            if h < HL:
                l_rdma[h].wait_recv()
                if h + 1 < HL:
                    l_rdma[h + 1].start()
                    pending_sends.append((h + 1, l_rdma[h + 1]))

            rho = N_DEV - 1 - h
            blk_t = lax.dot_general(wcom_r[h], xv, (((1,), (1,)), ((), ())),
                                    preferred_element_type=jnp.float32)
            ysend[rho] = blk_t
            y_rdma[rho].start()
            pending_sends.append((h, y_rdma[rho]))
            amax = jnp.maximum(amax, jnp.max(jnp.abs(blk_t)))
            if h < HL:
                rho = h + 1
                blk_t = lax.dot_general(wcom_l[h], xv, (((1,), (1,)), ((), ())),
                                        preferred_element_type=jnp.float32)
                ysend[rho] = blk_t
                y_rdma[rho].start()
                pending_sends.append((h, y_rdma[rho]))
                amax = jnp.maximum(amax, jnp.max(jnp.abs(blk_t)))
            drain_sends(h - 1)

        for s in range(LOG2_N):
            amax_s[s] = jnp.full((8, 128), amax, jnp.float32)
            peer = my ^ (1 << s)
            rdma = pltpu.make_async_remote_copy(
                src_ref=amax_s.at[s],
                dst_ref=amax_r.at[s],
                send_sem=red_send.at[s],
                recv_sem=red_recv.at[s],
                device_id=(peer,),
                device_id_type=pl.DeviceIdType.MESH,
            )
            rdma.start()
            rdma.wait()
            amax = jnp.maximum(amax, amax_r[s, 0, 0])

        for rho in range(1, N_DEV):
            y_rdma[rho].wait_recv()
            origin = lax.rem(my + N_DEV - rho, N_DEV)
            out_ref[pl.ds(origin * m_per, m_per), :] = jnp.swapaxes(
                yrecv[rho], 0, 1)

        drain_sends(HR + 1)

        scale = amax / 127.0
        y = out_ref[:, :]
        q = jnp.clip(jnp.round(y / scale), -127.0, 127.0)
        out_ref[:, :] = q * scale

        @functools.partial(pl.run_scoped, exit_sem=pltpu.SemaphoreType.REGULAR)
        def _(exit_sem):
            pl.semaphore_signal(exit_sem, inc=1, device_id=(left,),
                                device_id_type=pl.DeviceIdType.MESH)
            pl.semaphore_signal(exit_sem, inc=1, device_id=(right,),
                                device_id_type=pl.DeviceIdType.MESH)
            pl.semaphore_wait(exit_sem, 2)

    return pl.pallas_call(
        body,
        out_shape=jax.ShapeDtypeStruct((m_total, n_local), jnp.float32),
        in_specs=[
            pl.BlockSpec(memory_space=pltpu.VMEM),
            pl.BlockSpec(memory_space=pltpu.VMEM),
        ],
        out_specs=pl.BlockSpec(memory_space=pltpu.VMEM),
        scratch_shapes=[
            pltpu.VMEM((m_per, k), jnp.bfloat16),
            pltpu.VMEM((n_local, k), jnp.bfloat16),
            pltpu.VMEM((HR, n_local, k), jnp.bfloat16),
            pltpu.VMEM((HL, n_local, k), jnp.bfloat16),
            pltpu.VMEM((N_DEV, n_local, m_per), jnp.float32),
            pltpu.VMEM((N_DEV, n_local, m_per), jnp.float32),
            pltpu.VMEM((LOG2_N, 8, 128), jnp.float32),
            pltpu.VMEM((LOG2_N, 8, 128), jnp.float32),
            pltpu.SemaphoreType.DMA((HR,)),
            pltpu.SemaphoreType.DMA((HR,)),
            pltpu.SemaphoreType.DMA((HL,)),
            pltpu.SemaphoreType.DMA((HL,)),
            pltpu.SemaphoreType.DMA((N_DEV,)),
            pltpu.SemaphoreType.DMA((N_DEV,)),
            pltpu.SemaphoreType.DMA((LOG2_N,)),
            pltpu.SemaphoreType.DMA((LOG2_N,)),
        ],
        compiler_params=pltpu.CompilerParams(
            collective_id=0,
            vmem_limit_bytes=100 * 1024 * 1024,
            skip_device_barrier=True,
        ),
    )(x, w_mat)
